# Initial kernel scaffold; baseline (speedup 1.0000x reference)
import jax
import jax.numpy as jnp
from jax.experimental import pallas as pl


def kernel(x, edge_index, W_l1, b1, W_r1, g1, bt1, W_l2, b2, W_r2, g2, bt2, W_l3, b3, W_r3):
    raise NotImplementedError("write your pallas kernel here")


# speedup vs baseline: 4.0302x; 4.0302x over previous
import functools

import jax
import jax.numpy as jnp
from jax import lax
from jax.experimental import pallas as pl
from jax.experimental.pallas import tpu as pltpu
from jax.experimental.pallas import tpu_sc as plsc

N = 10000
NP = 10240
E = 320000
NC = 2
NS = 16
K = 80
RPS = NP // NS

_mesh = plsc.VectorSubcoreMesh(
    core_axis_name="c", subcore_axis_name="s", num_cores=NC, num_subcores=NS)


def _make_sc_segsum(width, split_features):
    if split_features:
        eps = E // NS
    else:
        eps = E // (NC * NS)
    nblk = eps // K

    @functools.partial(
        pl.kernel,
        out_type=jax.ShapeDtypeStruct((2 * NP, width), jnp.float32),
        mesh=_mesh,
        scratch_types=[
            pltpu.VMEM((K,), jnp.int32),
            pltpu.VMEM((K,), jnp.int32),
            pltpu.VMEM((K, width), jnp.float32),
            pltpu.VMEM_SHARED((NP, width), jnp.float32),
            pltpu.SemaphoreType.DMA,
        ],
        compiler_params=pltpu.CompilerParams(use_tc_tiling_on_sc=False),
    )
    def k(table_h, src_h, dst_h, z_h, out_h, src_v, dst_v, rows_v, acc, sem):
        cid = lax.axis_index("c")
        sid = lax.axis_index("s")
        rs = sid * RPS

        pltpu.sync_copy(z_h.at[pl.ds(rs, RPS)], acc.at[pl.ds(rs, RPS)])
        plsc.subcore_barrier()

        if split_features:
            base = sid * eps
            idx_off = cid * N
        else:
            base = (cid * NS + sid) * eps
            idx_off = 0

        def body(i, carry):
            off = pl.multiple_of(base + i * K, 8)
            pltpu.sync_copy(src_h.at[pl.ds(off, K)], src_v)
            pltpu.sync_copy(dst_h.at[pl.ds(off, K)], dst_v)
            if split_features:
                for j in range(K // 16):
                    sl = pl.ds(j * 16, 16)
                    src_v[sl] = src_v[sl] + idx_off
            pltpu.async_copy(table_h.at[src_v], rows_v, sem).wait()
            pltpu.sync_copy(rows_v, acc.at[dst_v], add=True)
            return carry

        lax.fori_loop(0, nblk, body, 0, unroll=False)
        plsc.subcore_barrier()

        out_row = cid * NP + rs
        pltpu.sync_copy(acc.at[pl.ds(rs, RPS)], out_h.at[pl.ds(out_row, RPS)])

    return k


_sc_segsum_80 = _make_sc_segsum(80, split_features=True)
_sc_segsum_128 = _make_sc_segsum(128, split_features=True)
_sc_segsum_16 = _make_sc_segsum(16, split_features=False)


def _tc_layer1(s1sum, x, wl, b, wr, g, bt):
    def body(s_ref, x_ref, wl_ref, b_ref, wr_ref, g_ref, bt_ref,
             h1a_ref, h1b_ref, dinv_ref):
        deg = jnp.maximum(s_ref[0:N, 64:65], 1.0)
        dinv = 1.0 / deg
        agg = jnp.concatenate(
            [s_ref[0:N, 0:64], s_ref[NP:NP + N, 0:64]], axis=1) * dinv
        h = (lax.dot_general(agg, wl_ref[...], (((1,), (1,)), ((), ())),
                             preferred_element_type=jnp.float32)
             + b_ref[...]
             + lax.dot_general(x_ref[...], wr_ref[...], (((1,), (1,)), ((), ())),
                               preferred_element_type=jnp.float32))
        mu = jnp.mean(h, axis=0, keepdims=True)
        hc = h - mu
        var = jnp.mean(hc * hc, axis=0, keepdims=True)
        hn = jnp.maximum(g_ref[...] * hc * lax.rsqrt(var + 1e-5) + bt_ref[...], 0.0)
        h1a_ref[...] = hn[:, :128]
        h1b_ref[...] = hn[:, 128:]
        dinv_ref[...] = dinv

    return pl.pallas_call(
        body,
        out_shape=(jax.ShapeDtypeStruct((N, 128), jnp.float32),
                   jax.ShapeDtypeStruct((N, 128), jnp.float32),
                   jax.ShapeDtypeStruct((N, 1), jnp.float32)),
    )(s1sum, x, wl, b, wr, g, bt)


def _tc_layer2(h1a, h1b, s2sum, dinv, wl, b, wr, g, bt, wl3, wr3):
    def body(h1a_ref, h1b_ref, s_ref, dinv_ref, wl_ref, b_ref,
             wr_ref, g_ref, bt_ref, wl3_ref, wr3_ref, y16_ref, root_ref):
        dinv = dinv_ref[...]
        agg = jnp.concatenate(
            [s_ref[0:N, :], s_ref[NP:NP + N, :]], axis=1) * dinv
        h1 = jnp.concatenate([h1a_ref[...], h1b_ref[...]], axis=1)
        h = (lax.dot_general(agg, wl_ref[...], (((1,), (1,)), ((), ())),
                             preferred_element_type=jnp.float32)
             + b_ref[...]
             + lax.dot_general(h1, wr_ref[...], (((1,), (1,)), ((), ())),
                               preferred_element_type=jnp.float32))
        mu = jnp.mean(h, axis=0, keepdims=True)
        hc = h - mu
        var = jnp.mean(hc * hc, axis=0, keepdims=True)
        h2 = jnp.maximum(g_ref[...] * hc * lax.rsqrt(var + 1e-5) + bt_ref[...], 0.0)
        y = lax.dot_general(h2, wl3_ref[...], (((1,), (1,)), ((), ())),
                            preferred_element_type=jnp.float32)
        y16_ref[...] = jnp.broadcast_to(y, (N, 16))
        root_ref[...] = lax.dot_general(h2, wr3_ref[...], (((1,), (1,)), ((), ())),
                                        preferred_element_type=jnp.float32)

    return pl.pallas_call(
        body,
        out_shape=(jax.ShapeDtypeStruct((N, 16), jnp.float32),
                   jax.ShapeDtypeStruct((N, 1), jnp.float32)),
    )(h1a, h1b, s2sum, dinv, wl, b, wr, g, bt, wl3, wr3)


def _tc_final(s3sum, dinv, root, b3):
    def body(s_ref, dinv_ref, root_ref, b3_ref, out_ref):
        s = s_ref[0:N, 0:1] + s_ref[NP:NP + N, 0:1]
        out_ref[...] = s * dinv_ref[...] + b3_ref[...] + root_ref[...]

    return pl.pallas_call(
        body,
        out_shape=jax.ShapeDtypeStruct((N, 1), jnp.float32),
    )(s3sum, dinv, root, b3)


def kernel(x, edge_index, W_l1, b1, W_r1, g1, bt1,
           W_l2, b2, W_r2, g2, bt2, W_l3, b3, W_r3):
    src = edge_index[0]
    dst = edge_index[1]

    ones16 = jnp.ones((N, 16), jnp.float32)
    x01 = jnp.concatenate(
        [jnp.concatenate([x[:, :64], ones16], axis=1),
         jnp.concatenate([x[:, 64:], ones16], axis=1)], axis=0)

    z80 = jnp.zeros((NP, 80), jnp.float32)
    s1sum = _sc_segsum_80(x01, src, dst, z80)

    h1a, h1b, dinv = _tc_layer1(
        s1sum, x, W_l1, b1.reshape(1, -1), W_r1,
        g1.reshape(1, -1), bt1.reshape(1, -1))

    h1s = jnp.concatenate([h1a, h1b], axis=0)
    z128 = jnp.zeros((NP, 128), jnp.float32)
    s2sum = _sc_segsum_128(h1s, src, dst, z128)

    y16, root = _tc_layer2(
        h1a, h1b, s2sum, dinv, W_l2, b2.reshape(1, -1), W_r2,
        g2.reshape(1, -1), bt2.reshape(1, -1), W_l3, W_r3)

    z16 = jnp.zeros((NP, 16), jnp.float32)
    s3sum = _sc_segsum_16(y16, src, dst, z16)

    return _tc_final(s3sum, dinv, root, b3.reshape(1, 1))
